# baseline (speedup 1.0000x reference)
import functools
import math

import jax
import jax.numpy as jnp
import numpy as np
from jax import lax
from jax.experimental import pallas as pl
from jax.experimental.pallas import tpu as pltpu
from jax.experimental.pallas import tpu_sc as plsc

N_IDX = 65536
OUT_FLAT = 6 * 256 * 256
NW = 32
PER_W = OUT_FLAT // NW
CHUNK = 2048
NCHUNK = N_IDX // CHUNK
TBL = 6 * 4096
SENT = 0x7FFFFFFF


LISTP = PER_W + 16
LROWS = PER_W // 16


def _sc_scatter(p5, p6, p7, p8, t1, t2, t3, p1f):
    mesh = plsc.VectorSubcoreMesh(core_axis_name="c", subcore_axis_name="s")
    f32 = jnp.float32
    i32 = jnp.int32
    out_sd = jax.ShapeDtypeStruct((OUT_FLAT,), f32)

    @functools.partial(
        pl.kernel, mesh=mesh,
        out_type=(out_sd, out_sd, out_sd),
        compiler_params=pltpu.CompilerParams(needs_layout_passes=False),
        scratch_types=[
            pltpu.VMEM((PER_W,), f32),
            pltpu.VMEM((PER_W,), f32),
            pltpu.VMEM((PER_W,), f32),
            pltpu.VMEM((PER_W,), i32),
            pltpu.VMEM((2 * CHUNK,), i32),
            pltpu.VMEM((2 * CHUNK,), i32),
            pltpu.VMEM((2 * CHUNK,), i32),
            pltpu.VMEM((2 * CHUNK,), i32),
            pltpu.VMEM((LISTP,), i32),
            pltpu.VMEM((LISTP,), i32),
            pltpu.VMEM((LISTP,), f32),
            pltpu.VMEM((LISTP,), f32),
            pltpu.VMEM((LISTP,), f32),
            pltpu.SemaphoreType.DMA((2,)),
            pltpu.SemaphoreType.DMA,
        ],
    )
    def k(p5_h, p6_h, p7_h, p8_h, t1_h, t2_h, t3_h, p1_h,
          o1_h, o2_h, o3_h,
          o1_v, o2_v, o3_v, own_v, i5_v, i6_v, i7_v, i8_v,
          soffl, gidxl, v1b, v2b, v3b, dsem, sem):
        wid = lax.axis_index("s") * 2 + lax.axis_index("c")
        lo = wid * PER_W
        pltpu.sync_copy(p1_h.at[pl.ds(lo, PER_W)], o1_v)
        pltpu.sync_copy(p1_h.at[pl.ds(lo, PER_W)], o2_v)
        pltpu.sync_copy(p1_h.at[pl.ds(lo, PER_W)], o3_v)
        lane = lax.iota(i32, 16)

        def init_body(vi, carry):
            own_v[pl.ds(vi * 16, 16)] = jnp.full((16,), -1, i32)
            gidxl[pl.ds(vi * 16, 16)] = jnp.zeros((16,), i32)
            return carry

        lax.fori_loop(0, LROWS, init_body, 0, unroll=8)
        gidxl[pl.ds(PER_W, 16)] = jnp.zeros((16,), i32)

        lane_next = jnp.minimum(lane + 1, 15)[:, None]
        last_lane = lane == 15
        gdn = lax.GatherDimensionNumbers(
            offset_dims=(), collapsed_slice_dims=(0,), start_index_map=(0,))

        def stage(ci):
            par = ci & 1
            base = ci * CHUNK
            pltpu.async_copy(p5_h.at[pl.ds(base, CHUNK)], i5_v.at[pl.ds(par * CHUNK, CHUNK)],
                             dsem.at[par])
            pltpu.async_copy(p6_h.at[pl.ds(base, CHUNK)], i6_v.at[pl.ds(par * CHUNK, CHUNK)],
                             dsem.at[par])
            pltpu.async_copy(p7_h.at[pl.ds(base, CHUNK)], i7_v.at[pl.ds(par * CHUNK, CHUNK)],
                             dsem.at[par])
            pltpu.async_copy(p8_h.at[pl.ds(base, CHUNK)], i8_v.at[pl.ds(par * CHUNK, CHUNK)],
                             dsem.at[par])

        def stage_wait(ci):
            par = ci & 1
            base = ci * CHUNK
            pltpu.make_async_copy(p5_h.at[pl.ds(base, CHUNK)], i5_v.at[pl.ds(par * CHUNK, CHUNK)],
                                  dsem.at[par]).wait()
            pltpu.make_async_copy(p6_h.at[pl.ds(base, CHUNK)], i6_v.at[pl.ds(par * CHUNK, CHUNK)],
                                  dsem.at[par]).wait()
            pltpu.make_async_copy(p7_h.at[pl.ds(base, CHUNK)], i7_v.at[pl.ds(par * CHUNK, CHUNK)],
                                  dsem.at[par]).wait()
            pltpu.make_async_copy(p8_h.at[pl.ds(base, CHUNK)], i8_v.at[pl.ds(par * CHUNK, CHUNK)],
                                  dsem.at[par]).wait()

        stage(0)

        def chunk_body(ci, carry):
            par = ci & 1
            stage_wait(ci)

            @pl.when(ci + 1 < NCHUNK)
            def _():
                stage(ci + 1)

            @plsc.parallel_loop(0, CHUNK // 16, unroll=8)
            def compute_body(vi):
                off = vi * 16
                v5 = i5_v[pl.ds(par * CHUNK + off, 16)]
                v6 = i6_v[pl.ds(par * CHUNK + off, 16)]
                v7 = i7_v[pl.ds(par * CHUNK + off, 16)]
                v8 = i8_v[pl.ds(par * CHUNK + off, 16)]
                sidx = v5 * 65536 + v7 * 256 + v8
                gidx = v5 * 4096 + v6
                m = (sidx >= lo) & (sidx < lo + PER_W)
                key = jnp.where(m, sidx * 16 + lane, SENT)
                skey, sgid = plsc.sort_key_val(key, gidx)
                nxt = lax.gather(skey, lane_next, gdn, (1,),
                                 mode=lax.GatherScatterMode.PROMISE_IN_BOUNDS)
                shi = skey >> 4
                keep = ((shi != (nxt >> 4)) | last_lane) & (skey != SENT)
                packed = jnp.where(keep, ((shi - lo) << 15) | sgid, -1)
                v1b[pl.ds(off, 16)] = plsc.bitcast(packed, jnp.float32)

            def commit_body(vi, carry2):
                off = vi * 16
                pk = plsc.bitcast(v1b[pl.ds(off, 16)], jnp.int32)
                plsc.store_scatter(own_v, [pk >> 15], pk & 0x7FFF,
                                   mask=pk >= 0)
                return carry2

            return lax.fori_loop(0, CHUNK // 16, commit_body, carry,
                                 unroll=16)

        lax.fori_loop(0, NCHUNK, chunk_body, 0)

        def compact_body(vi, cur):
            w = own_v[pl.ds(vi * 16, 16)]
            hit = w >= 0
            plsc.store_compressed(soffl.at[pl.ds(cur, 16)],
                                  lane + vi * 16, mask=hit)
            plsc.store_compressed(gidxl.at[pl.ds(cur, 16)], w, mask=hit)
            return cur + jnp.max(plsc.all_reduce_population_count(hit))

        kcnt = lax.fori_loop(0, LROWS, compact_body, 0)

        rows = (kcnt + 127) >> 7

        def gather_body(r, carry):
            rb = r * 128
            idxs = gidxl.at[pl.ds(rb, 128)]
            pltpu.async_copy(t1_h.at[idxs], v1b.at[pl.ds(rb, 128)], sem)
            pltpu.async_copy(t2_h.at[idxs], v2b.at[pl.ds(rb, 128)], sem)
            pltpu.async_copy(t3_h.at[idxs], v3b.at[pl.ds(rb, 128)], sem)
            pltpu.make_async_copy(t1_h.at[idxs], v1b.at[pl.ds(rb, 128)],
                                  sem).wait()
            pltpu.make_async_copy(t2_h.at[idxs], v2b.at[pl.ds(rb, 128)],
                                  sem).wait()
            pltpu.make_async_copy(t3_h.at[idxs], v3b.at[pl.ds(rb, 128)],
                                  sem).wait()
            return carry

        lax.fori_loop(0, rows, gather_body, 0)

        def deliver_body(vi, carry):
            base = vi * 16
            valid = (lane + base) < kcnt
            so = soffl[pl.ds(base, 16)]
            plsc.store_scatter(o1_v, [so], v1b[pl.ds(base, 16)], mask=valid)
            plsc.store_scatter(o2_v, [so], v2b[pl.ds(base, 16)], mask=valid)
            plsc.store_scatter(o3_v, [so], v3b[pl.ds(base, 16)], mask=valid)
            return carry

        lax.fori_loop(0, (kcnt + 15) >> 4, deliver_body, 0)

        pltpu.sync_copy(o1_v, o1_h.at[pl.ds(lo, PER_W)])
        pltpu.sync_copy(o2_v, o2_h.at[pl.ds(lo, PER_W)])
        pltpu.sync_copy(o3_v, o3_h.at[pl.ds(lo, PER_W)])

    return k(p5, p6, p7, p8, t1, t2, t3, p1f)


def _tc_chain_body(ip1_ref, ip2_ref, ip3_ref, p3_ref, x_ref, scal_ref,
                   cos_ref, sin_ref, re_ref, im_ref):
    t = jax.lax.broadcasted_iota(jnp.int32, (1, 256), 1).astype(jnp.float32) * (2.0 * math.pi)

    def env(freq, phase):
        s = jnp.sin(t * freq + phase)
        return s * s * 0.1 + 0.95

    e1 = env(scal_ref[0, 0], scal_ref[0, 1])
    e2 = env(scal_ref[0, 2], scal_ref[0, 3])
    e3 = env(scal_ref[0, 4], scal_ref[0, 5])
    cosm = cos_ref[...]
    sinm = sin_ref[...]

    for b in range(6):
        m = p3_ref[b] * 0.975
        a1 = ip1_ref[b] + m
        a2 = ip2_ref[b] + m
        a3 = ip3_ref[b] + m
        x = x_ref[:, b, :]
        v3 = jnp.dot(x, a1, preferred_element_type=jnp.float32)
        d = v3 / e1
        v7 = jnp.dot(d, a2, preferred_element_type=jnp.float32)
        m12 = v7 * e2
        v11 = jnp.dot(m12, a3, preferred_element_type=jnp.float32)
        d1 = v11 / e3
        re_ref[:, b, :] = jnp.dot(d1, cosm, preferred_element_type=jnp.float32)
        im_ref[:, b, :] = jnp.dot(d1, sinm, preferred_element_type=jnp.float32)


def _tc_chain(ip1, ip2, ip3, p3, x, scal, cosm, sinm):
    out_sd = jax.ShapeDtypeStruct((12, 6, 129), jnp.float32)
    return pl.pallas_call(
        _tc_chain_body,
        out_shape=(out_sd, out_sd),
        in_specs=[
            pl.BlockSpec(memory_space=pltpu.VMEM),
            pl.BlockSpec(memory_space=pltpu.VMEM),
            pl.BlockSpec(memory_space=pltpu.VMEM),
            pl.BlockSpec(memory_space=pltpu.VMEM),
            pl.BlockSpec(memory_space=pltpu.VMEM),
            pl.BlockSpec(memory_space=pltpu.SMEM),
            pl.BlockSpec(memory_space=pltpu.VMEM),
            pl.BlockSpec(memory_space=pltpu.VMEM),
        ],
    )(ip1, ip2, ip3, p3, x, scal, cosm, sinm)


def kernel(primals_1, primals_2, primals_3, primals_4, primals_5, primals_6,
           primals_7, primals_8, primals_9, primals_10, primals_11,
           primals_12, primals_13, primals_14, primals_15, primals_16,
           primals_17, primals_18, primals_19, primals_20, primals_21,
           primals_22, primals_23, primals_24, primals_25, primals_26,
           primals_27, primals_28, primals_29, primals_30, primals_31,
           primals_32, primals_33, primals_34, primals_35, primals_36,
           primals_37):
    p5 = primals_5.astype(jnp.int32)
    p6 = primals_6.astype(jnp.int32)
    p7 = primals_7.astype(jnp.int32)
    p8 = primals_8.astype(jnp.int32)

    ip1f, ip2f, ip3f = _sc_scatter(
        p5, p6, p7, p8,
        primals_4.reshape(-1), primals_11.reshape(-1),
        primals_14.reshape(-1), primals_1.reshape(-1))
    ip1 = ip1f.reshape(6, 256, 256)
    ip2 = ip2f.reshape(6, 256, 256)
    ip3 = ip3f.reshape(6, 256, 256)

    scal = jnp.stack([primals_9, primals_10, primals_12, primals_13,
                      primals_15, primals_16, primals_16, primals_16]
                     ).reshape(1, 8).astype(jnp.float32)

    i = np.arange(256, dtype=np.float64)[:, None]
    j = np.arange(129, dtype=np.float64)[None, :]
    ang = (2.0 * math.pi / 256.0) * i * j
    cosm = jnp.asarray(np.cos(ang), dtype=jnp.float32)
    sinm = jnp.asarray(-np.sin(ang), dtype=jnp.float32)

    re, im = _tc_chain(ip1, ip2, ip3, primals_3, primals_2, scal, cosm, sinm)
    return jnp.stack((re, im), axis=-1)


# speedup vs baseline: 1.0234x; 1.0234x over previous
import functools
import math

import jax
import jax.numpy as jnp
import numpy as np
from jax import lax
from jax.experimental import pallas as pl
from jax.experimental.pallas import tpu as pltpu
from jax.experimental.pallas import tpu_sc as plsc

N_IDX = 65536
OUT_FLAT = 6 * 256 * 256
NW = 32
PER_W = OUT_FLAT // NW
CHUNK = 2048
NCHUNK = N_IDX // CHUNK
TBL = 6 * 4096
SENT = 0x7FFFFFFF


LISTP = PER_W + 16
LROWS = PER_W // 16


def _sc_scatter(p5, p6, p7, p8, t1, t2, t3, p1f):
    mesh = plsc.VectorSubcoreMesh(core_axis_name="c", subcore_axis_name="s")
    f32 = jnp.float32
    i32 = jnp.int32
    out_sd = jax.ShapeDtypeStruct((OUT_FLAT,), f32)

    @functools.partial(
        pl.kernel, mesh=mesh,
        out_type=(out_sd, out_sd, out_sd),
        compiler_params=pltpu.CompilerParams(needs_layout_passes=False),
        scratch_types=[
            pltpu.VMEM((PER_W,), f32),
            pltpu.VMEM((PER_W,), f32),
            pltpu.VMEM((PER_W,), f32),
            pltpu.VMEM((PER_W,), i32),
            pltpu.VMEM((2 * CHUNK,), i32),
            pltpu.VMEM((2 * CHUNK,), i32),
            pltpu.VMEM((2 * CHUNK,), i32),
            pltpu.VMEM((2 * CHUNK,), i32),
            pltpu.VMEM((LISTP,), i32),
            pltpu.VMEM((LISTP,), i32),
            pltpu.VMEM((LISTP,), f32),
            pltpu.VMEM((LISTP,), f32),
            pltpu.VMEM((LISTP,), f32),
            pltpu.SemaphoreType.DMA((2,)),
            pltpu.SemaphoreType.DMA,
        ],
    )
    def k(p5_h, p6_h, p7_h, p8_h, t1_h, t2_h, t3_h, p1_h,
          o1_h, o2_h, o3_h,
          o1_v, o2_v, o3_v, own_v, i5_v, i6_v, i7_v, i8_v,
          soffl, gidxl, v1b, v2b, v3b, dsem, sem):
        wid = lax.axis_index("s") * 2 + lax.axis_index("c")
        lo = wid * PER_W
        pltpu.sync_copy(p1_h.at[pl.ds(lo, PER_W)], o1_v)
        pltpu.sync_copy(p1_h.at[pl.ds(lo, PER_W)], o2_v)
        pltpu.sync_copy(p1_h.at[pl.ds(lo, PER_W)], o3_v)
        lane = lax.iota(i32, 16)

        def init_body(vi, carry):
            own_v[pl.ds(vi * 16, 16)] = jnp.full((16,), -1, i32)
            gidxl[pl.ds(vi * 16, 16)] = jnp.zeros((16,), i32)
            return carry

        lax.fori_loop(0, LROWS, init_body, 0, unroll=8)
        gidxl[pl.ds(PER_W, 16)] = jnp.zeros((16,), i32)

        lane_next = jnp.minimum(lane + 1, 15)[:, None]
        last_lane = lane == 15
        gdn = lax.GatherDimensionNumbers(
            offset_dims=(), collapsed_slice_dims=(0,), start_index_map=(0,))

        def stage(ci):
            par = ci & 1
            base = ci * CHUNK
            pltpu.async_copy(p5_h.at[pl.ds(base, CHUNK)], i5_v.at[pl.ds(par * CHUNK, CHUNK)],
                             dsem.at[par])
            pltpu.async_copy(p6_h.at[pl.ds(base, CHUNK)], i6_v.at[pl.ds(par * CHUNK, CHUNK)],
                             dsem.at[par])
            pltpu.async_copy(p7_h.at[pl.ds(base, CHUNK)], i7_v.at[pl.ds(par * CHUNK, CHUNK)],
                             dsem.at[par])
            pltpu.async_copy(p8_h.at[pl.ds(base, CHUNK)], i8_v.at[pl.ds(par * CHUNK, CHUNK)],
                             dsem.at[par])

        def stage_wait(ci):
            par = ci & 1
            base = ci * CHUNK
            pltpu.make_async_copy(p5_h.at[pl.ds(base, CHUNK)], i5_v.at[pl.ds(par * CHUNK, CHUNK)],
                                  dsem.at[par]).wait()
            pltpu.make_async_copy(p6_h.at[pl.ds(base, CHUNK)], i6_v.at[pl.ds(par * CHUNK, CHUNK)],
                                  dsem.at[par]).wait()
            pltpu.make_async_copy(p7_h.at[pl.ds(base, CHUNK)], i7_v.at[pl.ds(par * CHUNK, CHUNK)],
                                  dsem.at[par]).wait()
            pltpu.make_async_copy(p8_h.at[pl.ds(base, CHUNK)], i8_v.at[pl.ds(par * CHUNK, CHUNK)],
                                  dsem.at[par]).wait()

        stage(0)

        def chunk_body(ci, carry):
            par = ci & 1
            stage_wait(ci)

            @pl.when(ci + 1 < NCHUNK)
            def _():
                stage(ci + 1)

            @plsc.parallel_loop(0, CHUNK // 16, unroll=4)
            def compute_body(vi):
                off = vi * 16
                v5 = i5_v[pl.ds(par * CHUNK + off, 16)]
                v6 = i6_v[pl.ds(par * CHUNK + off, 16)]
                v7 = i7_v[pl.ds(par * CHUNK + off, 16)]
                v8 = i8_v[pl.ds(par * CHUNK + off, 16)]
                sidx = v5 * 65536 + v7 * 256 + v8
                gidx = v5 * 4096 + v6
                m = (sidx >= lo) & (sidx < lo + PER_W)
                key = jnp.where(m, sidx * 16 + lane, SENT)
                skey, sgid = plsc.sort_key_val(key, gidx)
                nxt = lax.gather(skey, lane_next, gdn, (1,),
                                 mode=lax.GatherScatterMode.PROMISE_IN_BOUNDS)
                shi = skey >> 4
                keep = ((shi != (nxt >> 4)) | last_lane) & (skey != SENT)
                packed = jnp.where(keep, ((shi - lo) << 15) | sgid, -1)
                v1b[pl.ds(off, 16)] = plsc.bitcast(packed, jnp.float32)

            def commit_body(vi, carry2):
                off = vi * 16
                pk = plsc.bitcast(v1b[pl.ds(off, 16)], jnp.int32)
                plsc.store_scatter(own_v, [pk >> 15], pk & 0x7FFF,
                                   mask=pk >= 0)
                return carry2

            return lax.fori_loop(0, CHUNK // 16, commit_body, carry,
                                 unroll=8)

        lax.fori_loop(0, NCHUNK, chunk_body, 0)

        def compact_body(vi, cur):
            w = own_v[pl.ds(vi * 16, 16)]
            hit = w >= 0
            plsc.store_compressed(soffl.at[pl.ds(cur, 16)],
                                  lane + vi * 16, mask=hit)
            plsc.store_compressed(gidxl.at[pl.ds(cur, 16)], w, mask=hit)
            return cur + jnp.max(plsc.all_reduce_population_count(hit))

        kcnt = lax.fori_loop(0, LROWS, compact_body, 0)

        rows = (kcnt + 127) >> 7

        def gather_body(r, carry):
            rb = r * 128
            idxs = gidxl.at[pl.ds(rb, 128)]
            pltpu.async_copy(t1_h.at[idxs], v1b.at[pl.ds(rb, 128)], sem)
            pltpu.async_copy(t2_h.at[idxs], v2b.at[pl.ds(rb, 128)], sem)
            pltpu.async_copy(t3_h.at[idxs], v3b.at[pl.ds(rb, 128)], sem)
            pltpu.make_async_copy(t1_h.at[idxs], v1b.at[pl.ds(rb, 128)],
                                  sem).wait()
            pltpu.make_async_copy(t2_h.at[idxs], v2b.at[pl.ds(rb, 128)],
                                  sem).wait()
            pltpu.make_async_copy(t3_h.at[idxs], v3b.at[pl.ds(rb, 128)],
                                  sem).wait()
            return carry

        lax.fori_loop(0, rows, gather_body, 0)

        def deliver_body(vi, carry):
            base = vi * 16
            valid = (lane + base) < kcnt
            so = soffl[pl.ds(base, 16)]
            plsc.store_scatter(o1_v, [so], v1b[pl.ds(base, 16)], mask=valid)
            plsc.store_scatter(o2_v, [so], v2b[pl.ds(base, 16)], mask=valid)
            plsc.store_scatter(o3_v, [so], v3b[pl.ds(base, 16)], mask=valid)
            return carry

        lax.fori_loop(0, (kcnt + 15) >> 4, deliver_body, 0)

        pltpu.sync_copy(o1_v, o1_h.at[pl.ds(lo, PER_W)])
        pltpu.sync_copy(o2_v, o2_h.at[pl.ds(lo, PER_W)])
        pltpu.sync_copy(o3_v, o3_h.at[pl.ds(lo, PER_W)])

    return k(p5, p6, p7, p8, t1, t2, t3, p1f)


def _tc_chain_body(ip1_ref, ip2_ref, ip3_ref, p3_ref, x_ref, scal_ref,
                   cos_ref, sin_ref, re_ref, im_ref):
    t = jax.lax.broadcasted_iota(jnp.int32, (1, 256), 1).astype(jnp.float32) * (2.0 * math.pi)

    def env(freq, phase):
        s = jnp.sin(t * freq + phase)
        return s * s * 0.1 + 0.95

    e1 = env(scal_ref[0, 0], scal_ref[0, 1])
    e2 = env(scal_ref[0, 2], scal_ref[0, 3])
    e3 = env(scal_ref[0, 4], scal_ref[0, 5])
    cosm = cos_ref[...]
    sinm = sin_ref[...]

    for b in range(6):
        m = p3_ref[b] * 0.975
        a1 = ip1_ref[b] + m
        a2 = ip2_ref[b] + m
        a3 = ip3_ref[b] + m
        x = x_ref[:, b, :]
        v3 = jnp.dot(x, a1, preferred_element_type=jnp.float32)
        d = v3 / e1
        v7 = jnp.dot(d, a2, preferred_element_type=jnp.float32)
        m12 = v7 * e2
        v11 = jnp.dot(m12, a3, preferred_element_type=jnp.float32)
        d1 = v11 / e3
        re_ref[:, b, :] = jnp.dot(d1, cosm, preferred_element_type=jnp.float32)
        im_ref[:, b, :] = jnp.dot(d1, sinm, preferred_element_type=jnp.float32)


def _tc_chain(ip1, ip2, ip3, p3, x, scal, cosm, sinm):
    out_sd = jax.ShapeDtypeStruct((12, 6, 129), jnp.float32)
    return pl.pallas_call(
        _tc_chain_body,
        out_shape=(out_sd, out_sd),
        in_specs=[
            pl.BlockSpec(memory_space=pltpu.VMEM),
            pl.BlockSpec(memory_space=pltpu.VMEM),
            pl.BlockSpec(memory_space=pltpu.VMEM),
            pl.BlockSpec(memory_space=pltpu.VMEM),
            pl.BlockSpec(memory_space=pltpu.VMEM),
            pl.BlockSpec(memory_space=pltpu.SMEM),
            pl.BlockSpec(memory_space=pltpu.VMEM),
            pl.BlockSpec(memory_space=pltpu.VMEM),
        ],
    )(ip1, ip2, ip3, p3, x, scal, cosm, sinm)


def kernel(primals_1, primals_2, primals_3, primals_4, primals_5, primals_6,
           primals_7, primals_8, primals_9, primals_10, primals_11,
           primals_12, primals_13, primals_14, primals_15, primals_16,
           primals_17, primals_18, primals_19, primals_20, primals_21,
           primals_22, primals_23, primals_24, primals_25, primals_26,
           primals_27, primals_28, primals_29, primals_30, primals_31,
           primals_32, primals_33, primals_34, primals_35, primals_36,
           primals_37):
    p5 = primals_5.astype(jnp.int32)
    p6 = primals_6.astype(jnp.int32)
    p7 = primals_7.astype(jnp.int32)
    p8 = primals_8.astype(jnp.int32)

    ip1f, ip2f, ip3f = _sc_scatter(
        p5, p6, p7, p8,
        primals_4.reshape(-1), primals_11.reshape(-1),
        primals_14.reshape(-1), primals_1.reshape(-1))
    ip1 = ip1f.reshape(6, 256, 256)
    ip2 = ip2f.reshape(6, 256, 256)
    ip3 = ip3f.reshape(6, 256, 256)

    scal = jnp.stack([primals_9, primals_10, primals_12, primals_13,
                      primals_15, primals_16, primals_16, primals_16]
                     ).reshape(1, 8).astype(jnp.float32)

    i = np.arange(256, dtype=np.float64)[:, None]
    j = np.arange(129, dtype=np.float64)[None, :]
    ang = (2.0 * math.pi / 256.0) * i * j
    cosm = jnp.asarray(np.cos(ang), dtype=jnp.float32)
    sinm = jnp.asarray(-np.sin(ang), dtype=jnp.float32)

    re, im = _tc_chain(ip1, ip2, ip3, primals_3, primals_2, scal, cosm, sinm)
    return jnp.stack((re, im), axis=-1)


# speedup vs baseline: 1.0368x; 1.0131x over previous
import functools
import math

import jax
import jax.numpy as jnp
import numpy as np
from jax import lax
from jax.experimental import pallas as pl
from jax.experimental.pallas import tpu as pltpu
from jax.experimental.pallas import tpu_sc as plsc

N_IDX = 65536
OUT_FLAT = 6 * 256 * 256
NW = 32
PER_W = OUT_FLAT // NW
CHUNK = 2048
NCHUNK = N_IDX // CHUNK
TBL = 6 * 4096
SENT = 0x7FFFFFFF


LISTP = PER_W + 16
LROWS = PER_W // 16


def _sc_scatter(p5, p6, p7, p8, t1, t2, t3, p1f):
    mesh = plsc.VectorSubcoreMesh(core_axis_name="c", subcore_axis_name="s")
    f32 = jnp.float32
    i32 = jnp.int32
    out_sd = jax.ShapeDtypeStruct((OUT_FLAT,), f32)

    @functools.partial(
        pl.kernel, mesh=mesh,
        out_type=(out_sd, out_sd, out_sd),
        compiler_params=pltpu.CompilerParams(needs_layout_passes=False),
        scratch_types=[
            pltpu.VMEM((PER_W,), f32),
            pltpu.VMEM((PER_W,), f32),
            pltpu.VMEM((PER_W,), f32),
            pltpu.VMEM((PER_W,), i32),
            pltpu.VMEM((2 * CHUNK,), i32),
            pltpu.VMEM((2 * CHUNK,), i32),
            pltpu.VMEM((2 * CHUNK,), i32),
            pltpu.VMEM((2 * CHUNK,), i32),
            pltpu.VMEM((LISTP,), i32),
            pltpu.VMEM((LISTP,), i32),
            pltpu.VMEM((LISTP,), f32),
            pltpu.VMEM((LISTP,), f32),
            pltpu.VMEM((LISTP,), f32),
            pltpu.SemaphoreType.DMA((2,)),
            pltpu.SemaphoreType.DMA,
        ],
    )
    def k(p5_h, p6_h, p7_h, p8_h, t1_h, t2_h, t3_h, p1_h,
          o1_h, o2_h, o3_h,
          o1_v, o2_v, o3_v, own_v, i5_v, i6_v, i7_v, i8_v,
          soffl, gidxl, v1b, v2b, v3b, dsem, sem):
        wid = lax.axis_index("s") * 2 + lax.axis_index("c")
        lo = wid * PER_W
        pltpu.async_copy(p1_h.at[pl.ds(lo, PER_W)], o1_v, sem)
        pltpu.async_copy(p1_h.at[pl.ds(lo, PER_W)], o2_v, sem)
        pltpu.async_copy(p1_h.at[pl.ds(lo, PER_W)], o3_v, sem)
        lane = lax.iota(i32, 16)

        def init_body(vi, carry):
            own_v[pl.ds(vi * 16, 16)] = jnp.full((16,), -1, i32)
            gidxl[pl.ds(vi * 16, 16)] = jnp.zeros((16,), i32)
            return carry

        lax.fori_loop(0, LROWS, init_body, 0, unroll=8)
        gidxl[pl.ds(PER_W, 16)] = jnp.zeros((16,), i32)

        lane_next = jnp.minimum(lane + 1, 15)[:, None]
        last_lane = lane == 15
        gdn = lax.GatherDimensionNumbers(
            offset_dims=(), collapsed_slice_dims=(0,), start_index_map=(0,))

        def stage(ci):
            par = ci & 1
            base = ci * CHUNK
            pltpu.async_copy(p5_h.at[pl.ds(base, CHUNK)], i5_v.at[pl.ds(par * CHUNK, CHUNK)],
                             dsem.at[par])
            pltpu.async_copy(p6_h.at[pl.ds(base, CHUNK)], i6_v.at[pl.ds(par * CHUNK, CHUNK)],
                             dsem.at[par])
            pltpu.async_copy(p7_h.at[pl.ds(base, CHUNK)], i7_v.at[pl.ds(par * CHUNK, CHUNK)],
                             dsem.at[par])
            pltpu.async_copy(p8_h.at[pl.ds(base, CHUNK)], i8_v.at[pl.ds(par * CHUNK, CHUNK)],
                             dsem.at[par])

        def stage_wait(ci):
            par = ci & 1
            base = ci * CHUNK
            pltpu.make_async_copy(p5_h.at[pl.ds(base, CHUNK)], i5_v.at[pl.ds(par * CHUNK, CHUNK)],
                                  dsem.at[par]).wait()
            pltpu.make_async_copy(p6_h.at[pl.ds(base, CHUNK)], i6_v.at[pl.ds(par * CHUNK, CHUNK)],
                                  dsem.at[par]).wait()
            pltpu.make_async_copy(p7_h.at[pl.ds(base, CHUNK)], i7_v.at[pl.ds(par * CHUNK, CHUNK)],
                                  dsem.at[par]).wait()
            pltpu.make_async_copy(p8_h.at[pl.ds(base, CHUNK)], i8_v.at[pl.ds(par * CHUNK, CHUNK)],
                                  dsem.at[par]).wait()

        stage(0)

        def chunk_body(ci, carry):
            par = ci & 1
            stage_wait(ci)

            @pl.when(ci + 1 < NCHUNK)
            def _():
                stage(ci + 1)

            @plsc.parallel_loop(0, CHUNK // 16, unroll=4)
            def compute_body(vi):
                off = vi * 16
                v5 = i5_v[pl.ds(par * CHUNK + off, 16)]
                v6 = i6_v[pl.ds(par * CHUNK + off, 16)]
                v7 = i7_v[pl.ds(par * CHUNK + off, 16)]
                v8 = i8_v[pl.ds(par * CHUNK + off, 16)]
                sidx = v5 * 65536 + v7 * 256 + v8
                gidx = v5 * 4096 + v6
                m = (sidx >= lo) & (sidx < lo + PER_W)
                key = jnp.where(m, sidx * 16 + lane, SENT)
                skey, sgid = plsc.sort_key_val(key, gidx)
                nxt = lax.gather(skey, lane_next, gdn, (1,),
                                 mode=lax.GatherScatterMode.PROMISE_IN_BOUNDS)
                shi = skey >> 4
                keep = ((shi != (nxt >> 4)) | last_lane) & (skey != SENT)
                packed = jnp.where(keep, ((shi - lo) << 15) | sgid, -1)
                v1b[pl.ds(off, 16)] = plsc.bitcast(packed, jnp.float32)

            def commit_body(vi, carry2):
                off = vi * 16
                pk = plsc.bitcast(v1b[pl.ds(off, 16)], jnp.int32)
                plsc.store_scatter(own_v, [pk >> 15], pk & 0x7FFF,
                                   mask=pk >= 0)
                return carry2

            return lax.fori_loop(0, CHUNK // 16, commit_body, carry,
                                 unroll=8)

        lax.fori_loop(0, NCHUNK, chunk_body, 0)

        def compact_body(vi, cur):
            w = own_v[pl.ds(vi * 16, 16)]
            hit = w >= 0
            plsc.store_compressed(soffl.at[pl.ds(cur, 16)],
                                  lane + vi * 16, mask=hit)
            plsc.store_compressed(gidxl.at[pl.ds(cur, 16)], w, mask=hit)
            return cur + jnp.max(plsc.all_reduce_population_count(hit))

        kcnt = lax.fori_loop(0, LROWS, compact_body, 0)
        pltpu.make_async_copy(p1_h.at[pl.ds(lo, PER_W)], o1_v, sem).wait()
        pltpu.make_async_copy(p1_h.at[pl.ds(lo, PER_W)], o2_v, sem).wait()
        pltpu.make_async_copy(p1_h.at[pl.ds(lo, PER_W)], o3_v, sem).wait()

        rows = (kcnt + 127) >> 7

        def gather_body(r, carry):
            rb = r * 128
            idxs = gidxl.at[pl.ds(rb, 128)]
            pltpu.async_copy(t1_h.at[idxs], v1b.at[pl.ds(rb, 128)], sem)
            pltpu.async_copy(t2_h.at[idxs], v2b.at[pl.ds(rb, 128)], sem)
            pltpu.async_copy(t3_h.at[idxs], v3b.at[pl.ds(rb, 128)], sem)
            pltpu.make_async_copy(t1_h.at[idxs], v1b.at[pl.ds(rb, 128)],
                                  sem).wait()
            pltpu.make_async_copy(t2_h.at[idxs], v2b.at[pl.ds(rb, 128)],
                                  sem).wait()
            pltpu.make_async_copy(t3_h.at[idxs], v3b.at[pl.ds(rb, 128)],
                                  sem).wait()
            return carry

        lax.fori_loop(0, rows, gather_body, 0)

        def deliver_body(vi, carry):
            base = vi * 16
            valid = (lane + base) < kcnt
            so = soffl[pl.ds(base, 16)]
            plsc.store_scatter(o1_v, [so], v1b[pl.ds(base, 16)], mask=valid)
            plsc.store_scatter(o2_v, [so], v2b[pl.ds(base, 16)], mask=valid)
            plsc.store_scatter(o3_v, [so], v3b[pl.ds(base, 16)], mask=valid)
            return carry

        lax.fori_loop(0, (kcnt + 15) >> 4, deliver_body, 0)

        pltpu.sync_copy(o1_v, o1_h.at[pl.ds(lo, PER_W)])
        pltpu.sync_copy(o2_v, o2_h.at[pl.ds(lo, PER_W)])
        pltpu.sync_copy(o3_v, o3_h.at[pl.ds(lo, PER_W)])

    return k(p5, p6, p7, p8, t1, t2, t3, p1f)


def _tc_chain_body(ip1_ref, ip2_ref, ip3_ref, p3_ref, x_ref, scal_ref,
                   cos_ref, sin_ref, re_ref, im_ref):
    t = jax.lax.broadcasted_iota(jnp.int32, (1, 256), 1).astype(jnp.float32) * (2.0 * math.pi)

    def env(freq, phase):
        s = jnp.sin(t * freq + phase)
        return s * s * 0.1 + 0.95

    e1 = env(scal_ref[0, 0], scal_ref[0, 1])
    e2 = env(scal_ref[0, 2], scal_ref[0, 3])
    e3 = env(scal_ref[0, 4], scal_ref[0, 5])
    cosm = cos_ref[...]
    sinm = sin_ref[...]

    for b in range(6):
        m = p3_ref[b] * 0.975
        a1 = ip1_ref[b] + m
        a2 = ip2_ref[b] + m
        a3 = ip3_ref[b] + m
        x = x_ref[:, b, :]
        v3 = jnp.dot(x, a1, preferred_element_type=jnp.float32)
        d = v3 / e1
        v7 = jnp.dot(d, a2, preferred_element_type=jnp.float32)
        m12 = v7 * e2
        v11 = jnp.dot(m12, a3, preferred_element_type=jnp.float32)
        d1 = v11 / e3
        re_ref[:, b, :] = jnp.dot(d1, cosm, preferred_element_type=jnp.float32)
        im_ref[:, b, :] = jnp.dot(d1, sinm, preferred_element_type=jnp.float32)


def _tc_chain(ip1, ip2, ip3, p3, x, scal, cosm, sinm):
    out_sd = jax.ShapeDtypeStruct((12, 6, 129), jnp.float32)
    return pl.pallas_call(
        _tc_chain_body,
        out_shape=(out_sd, out_sd),
        in_specs=[
            pl.BlockSpec(memory_space=pltpu.VMEM),
            pl.BlockSpec(memory_space=pltpu.VMEM),
            pl.BlockSpec(memory_space=pltpu.VMEM),
            pl.BlockSpec(memory_space=pltpu.VMEM),
            pl.BlockSpec(memory_space=pltpu.VMEM),
            pl.BlockSpec(memory_space=pltpu.SMEM),
            pl.BlockSpec(memory_space=pltpu.VMEM),
            pl.BlockSpec(memory_space=pltpu.VMEM),
        ],
    )(ip1, ip2, ip3, p3, x, scal, cosm, sinm)


def kernel(primals_1, primals_2, primals_3, primals_4, primals_5, primals_6,
           primals_7, primals_8, primals_9, primals_10, primals_11,
           primals_12, primals_13, primals_14, primals_15, primals_16,
           primals_17, primals_18, primals_19, primals_20, primals_21,
           primals_22, primals_23, primals_24, primals_25, primals_26,
           primals_27, primals_28, primals_29, primals_30, primals_31,
           primals_32, primals_33, primals_34, primals_35, primals_36,
           primals_37):
    p5 = primals_5.astype(jnp.int32)
    p6 = primals_6.astype(jnp.int32)
    p7 = primals_7.astype(jnp.int32)
    p8 = primals_8.astype(jnp.int32)

    ip1f, ip2f, ip3f = _sc_scatter(
        p5, p6, p7, p8,
        primals_4.reshape(-1), primals_11.reshape(-1),
        primals_14.reshape(-1), primals_1.reshape(-1))
    ip1 = ip1f.reshape(6, 256, 256)
    ip2 = ip2f.reshape(6, 256, 256)
    ip3 = ip3f.reshape(6, 256, 256)

    scal = jnp.stack([primals_9, primals_10, primals_12, primals_13,
                      primals_15, primals_16, primals_16, primals_16]
                     ).reshape(1, 8).astype(jnp.float32)

    i = np.arange(256, dtype=np.float64)[:, None]
    j = np.arange(129, dtype=np.float64)[None, :]
    ang = (2.0 * math.pi / 256.0) * i * j
    cosm = jnp.asarray(np.cos(ang), dtype=jnp.float32)
    sinm = jnp.asarray(-np.sin(ang), dtype=jnp.float32)

    re, im = _tc_chain(ip1, ip2, ip3, primals_3, primals_2, scal, cosm, sinm)
    return jnp.stack((re, im), axis=-1)
